# baseline (speedup 1.0000x reference)
import functools

import jax
import jax.numpy as jnp
from jax import lax
from jax.experimental import pallas as pl
from jax.experimental.pallas import tpu as pltpu
from jax.experimental.pallas import tpu_sc as plsc

B, N, DIM = 16, 576, 256
BN = B * N
K = 8192

KT = 1024
NKC = K // KT
GPC = KT // 128
TOK = 256
NTT = BN // TOK

NC, NS = 2, 16
NW = NC * NS
BPW = BN // NW
NCH, CH = 3, 96
RG = CH // 16

_MINOR = (((1,), (1,)), ((), ()))


def _assign_kernel(cb_ref, w_ref, x_ref, rows_ref, idx_ref, a_ref, b_ref,
                   loss_ref, cols_scr):
    i = pl.program_id(0)

    @pl.when(i < NKC)
    def _():
        cb = cb_ref[...]
        w = w_ref[...]
        icb = lax.dot_general(cb, w, _MINOR, preferred_element_type=jnp.float32)
        rn = jnp.sqrt(jnp.sum(icb * icb, axis=1, keepdims=True))
        rows_ref[...] = icb * (1.0 / jnp.clip(rn, 1e-12))
        icbt = lax.dot_general(w, cb, _MINOR, preferred_element_type=jnp.float32)
        cn = jnp.sqrt(jnp.sum(icbt * icbt, axis=0, keepdims=True))
        cols_scr[jnp.minimum(i, NKC - 1)] = icbt / jnp.clip(cn, 1e-12)

    @pl.when(i >= NKC)
    def _():
        xb = x_ref[...]
        nrm = jnp.sqrt(jnp.sum(xb * xb, axis=1, keepdims=True))
        xn = xb / jnp.clip(nrm, 1e-12)

        bv = jnp.full((TOK, 128), -jnp.inf, dtype=jnp.float32)
        bg = jnp.zeros((TOK, 128), dtype=jnp.int32)
        for c in range(NKC):
            sim = jnp.dot(xn, cols_scr[c], preferred_element_type=jnp.float32)
            for g in range(GPC):
                v = sim[:, g * 128:(g + 1) * 128]
                upd = v > bv
                bv = jnp.where(upd, v, bv)
                bg = jnp.where(upd, c * GPC + g, bg)

        lane = lax.broadcasted_iota(jnp.int32, (TOK, 128), 1)
        gidx = bg * 128 + lane
        m = jnp.max(bv, axis=1, keepdims=True)
        cand = jnp.where(bv == m, gidx, K)
        idx_ref[...] = jnp.min(cand, axis=1).reshape(1, 1, TOK)

        rinv = 1.0 / jnp.clip(nrm, 1e-12)
        un = nrm * rinv
        eu = un * un
        ss = eu + 2.0 * m + 1.0
        winv = 1.0 / jnp.clip(jnp.sqrt(ss), 1e-6)
        eww = (eu + m) * winv * winv
        alpha = rinv * (1.0 - 2.0 * eww)
        beta = 2.0 * (eu - eww)
        a_ref[...] = alpha.reshape(1, 1, TOK)
        b_ref[...] = beta.reshape(1, 1, TOK)

        part = (jnp.sum(eu - 2.0 * m + 1.0, axis=(0, 1), keepdims=True)
                * (1.25 / (BN * DIM)))

        @pl.when(i == NKC)
        def _():
            loss_ref[...] = jnp.zeros_like(part)

        loss_ref[...] += part


@functools.lru_cache(maxsize=1)
def _make_rot_gather():
    mesh = plsc.VectorSubcoreMesh(
        core_axis_name="c", subcore_axis_name="s",
        num_cores=NC, num_subcores=NS)

    @functools.partial(
        pl.kernel,
        mesh=mesh,
        compiler_params=pltpu.CompilerParams(
            use_tc_tiling_on_sc=False, needs_layout_passes=False),
        out_type=jax.ShapeDtypeStruct((NW, NCH, CH, DIM), jnp.float32),
        scratch_types=[
            pltpu.VMEM((NCH, CH), jnp.int32),
            pltpu.VMEM((NCH, CH), jnp.float32),
            pltpu.VMEM((NCH, CH), jnp.float32),
            pltpu.VMEM((CH, DIM), jnp.float32),
            pltpu.VMEM((CH, DIM), jnp.float32),
            pltpu.VMEM((CH, DIM), jnp.float32),
            pltpu.SemaphoreType.DMA,
            pltpu.SemaphoreType.DMA,
        ],
    )
    def _body(table_hbm, idx_hbm, x_hbm, a_hbm, b_hbm, out_hbm,
              idx_v, a_v, b_v, q_v, x_v, o_v, gsem, xsem):
        wid = lax.axis_index("s") * NC + lax.axis_index("c")
        pltpu.sync_copy(idx_hbm.at[wid], idx_v)
        pltpu.sync_copy(a_hbm.at[wid], a_v)
        pltpu.sync_copy(b_hbm.at[wid], b_v)
        for j in range(NCH):
            gcp = pltpu.async_copy(table_hbm.at[idx_v.at[j]], q_v, gsem)
            xcp = pltpu.async_copy(x_hbm.at[wid, j], x_v, xsem)
            gcp.wait()
            xcp.wait()
            for g in range(RG):
                ridx = jax.lax.iota(jnp.int32, 16) + g * 16
                av = a_v[j, pl.ds(g * 16, 16)]
                bvv = b_v[j, pl.ds(g * 16, 16)]

                def dbody(d, _):
                    didx = jnp.full((16,), d, dtype=jnp.int32)
                    xc = plsc.load_gather(x_v, [ridx, didx])
                    qc = plsc.load_gather(q_v, [ridx, didx])
                    plsc.store_scatter(o_v, [ridx, didx], av * xc + bvv * qc)
                    return 0

                lax.fori_loop(0, DIM, dbody, 0)
            pltpu.sync_copy(o_v, out_hbm.at[wid, j])

    return _body


def _rot_gather(table, idx3, x4, a3, b3):
    return _make_rot_gather()(table, idx3, x4, a3, b3)


def kernel(x, frozen_codebook, W):
    b, n, d = x.shape
    xf = x.reshape(b * n, d)

    rows, idx3, a3, b3, loss = pl.pallas_call(
        _assign_kernel,
        grid=(NKC + NTT,),
        in_specs=[
            pl.BlockSpec((KT, DIM), lambda i: (jnp.minimum(i, NKC - 1), 0)),
            pl.BlockSpec((DIM, DIM), lambda i: (0, 0)),
            pl.BlockSpec((TOK, DIM), lambda i: (jnp.maximum(i - NKC, 0), 0)),
        ],
        out_specs=[
            pl.BlockSpec((KT, DIM), lambda i: (jnp.minimum(i, NKC - 1), 0)),
            pl.BlockSpec((1, 1, TOK), lambda i: (jnp.maximum(i - NKC, 0), 0, 0)),
            pl.BlockSpec((1, 1, TOK), lambda i: (jnp.maximum(i - NKC, 0), 0, 0)),
            pl.BlockSpec((1, 1, TOK), lambda i: (jnp.maximum(i - NKC, 0), 0, 0)),
            pl.BlockSpec((1, 1), lambda i: (0, 0)),
        ],
        out_shape=[
            jax.ShapeDtypeStruct((K, DIM), jnp.float32),
            jax.ShapeDtypeStruct((NTT, 1, TOK), jnp.int32),
            jax.ShapeDtypeStruct((NTT, 1, TOK), jnp.float32),
            jax.ShapeDtypeStruct((NTT, 1, TOK), jnp.float32),
            jax.ShapeDtypeStruct((1, 1), jnp.float32),
        ],
        scratch_shapes=[pltpu.VMEM((NKC, DIM, KT), jnp.float32)],
    )(frozen_codebook, W, xf)

    indices = idx3.reshape(BN)
    rot4 = _rot_gather(
        rows,
        indices.reshape(NW, NCH, CH),
        xf.reshape(NW, NCH, CH, DIM),
        a3.reshape(NW, NCH, CH),
        b3.reshape(NW, NCH, CH),
    )

    return (rot4.reshape(b, n, d), indices.reshape(b, n), loss.reshape(()))


# speedup vs baseline: 2.5822x; 2.5822x over previous
import functools

import jax
import jax.numpy as jnp
from jax import lax
from jax.experimental import pallas as pl
from jax.experimental.pallas import tpu as pltpu
from jax.experimental.pallas import tpu_sc as plsc

B, N, DIM = 16, 576, 256
BN = B * N
K = 8192

KT = 1024
NKC = K // KT
GPC = KT // 128
TOK = 512
NTT = BN // TOK
TOK4 = 1152

NC, NS = 2, 16
NW = NC * NS
BPW = BN // NW
NCH, CH = 3, 96

_MINOR = (((1,), (1,)), ((), ()))


def _assign_kernel(cb_ref, w_ref, x_ref, rows_ref, idx_ref, cols_scr):
    i = pl.program_id(0)

    @pl.when(i < NKC)
    def _():
        cb = cb_ref[...]
        w = w_ref[...]
        icb = lax.dot_general(cb, w, _MINOR, preferred_element_type=jnp.float32)
        rn = jnp.sqrt(jnp.sum(icb * icb, axis=1, keepdims=True))
        rows_ref[...] = icb * (1.0 / jnp.clip(rn, 1e-12))
        icbt = lax.dot_general(w, cb, _MINOR, preferred_element_type=jnp.float32)
        cn = jnp.sqrt(jnp.sum(icbt * icbt, axis=0, keepdims=True))
        cols_scr[jnp.minimum(i, NKC - 1)] = icbt / jnp.clip(cn, 1e-12)

    @pl.when(i >= NKC)
    def _():
        xb = x_ref[...]
        nrm = jnp.sqrt(jnp.sum(xb * xb, axis=1, keepdims=True))
        xn = xb / jnp.clip(nrm, 1e-12)

        bv = jnp.full((TOK, 128), -jnp.inf, dtype=jnp.float32)
        bg = jnp.zeros((TOK, 128), dtype=jnp.int32)
        for c in range(NKC):
            sim = jnp.dot(xn, cols_scr[c], preferred_element_type=jnp.float32)
            for g in range(GPC):
                v = sim[:, g * 128:(g + 1) * 128]
                upd = v > bv
                bv = jnp.where(upd, v, bv)
                bg = jnp.where(upd, c * GPC + g, bg)

        lane = lax.broadcasted_iota(jnp.int32, (TOK, 128), 1)
        gidx = bg * 128 + lane
        m = jnp.max(bv, axis=1, keepdims=True)
        cand = jnp.where(bv == m, gidx, K)
        idx_ref[...] = jnp.min(cand, axis=1).reshape(1, 1, TOK)


def _rot_kernel(x_ref, q_ref, out_ref, loss_ref):
    i = pl.program_id(0)
    xb = x_ref[...]
    q = q_ref[...]
    a = jnp.sum(xb * xb, axis=1, keepdims=True)
    bq = jnp.sum(xb * q, axis=1, keepdims=True)
    c = jnp.sum(q * q, axis=1, keepdims=True)
    rinv = 1.0 / jnp.clip(jnp.sqrt(a), 1e-12)
    eu = a * rinv * rinv
    eq = bq * rinv
    ss = eu + 2.0 * eq + c
    winv = 1.0 / jnp.clip(jnp.sqrt(ss), 1e-6)
    eww = (eu + eq) * winv * winv
    alpha = rinv * (1.0 - 2.0 * eww)
    beta = 2.0 * (eu - eww)
    out_ref[...] = alpha * xb + beta * q
    part = (jnp.sum(eu - 2.0 * eq + c, axis=(0, 1), keepdims=True)
            * (1.25 / (BN * DIM)))

    @pl.when(i == 0)
    def _():
        loss_ref[...] = jnp.zeros_like(part)

    loss_ref[...] += part


@functools.lru_cache(maxsize=1)
def _make_gather():
    mesh = plsc.VectorSubcoreMesh(
        core_axis_name="c", subcore_axis_name="s",
        num_cores=NC, num_subcores=NS)

    @functools.partial(
        pl.kernel,
        mesh=mesh,
        out_type=jax.ShapeDtypeStruct((NW, NCH, CH, DIM), jnp.float32),
        scratch_types=[
            pltpu.VMEM((NCH, CH), jnp.int32),
            pltpu.VMEM((NCH, CH, DIM), jnp.float32),
            pltpu.SemaphoreType.DMA,
        ],
    )
    def _gather_body(table_hbm, idx_hbm, out_hbm, idx_v, rows_v, sem):
        wid = lax.axis_index("s") * NC + lax.axis_index("c")
        pltpu.sync_copy(idx_hbm.at[wid], idx_v)
        copies = [
            pltpu.async_copy(table_hbm.at[idx_v.at[j]], rows_v.at[j], sem)
            for j in range(NCH)
        ]
        for c in copies:
            c.wait()
        pltpu.sync_copy(rows_v, out_hbm.at[wid])

    return _gather_body


def _gather_kernel(table, idx3):
    return _make_gather()(table, idx3)


def kernel(x, frozen_codebook, W):
    b, n, d = x.shape
    xf = x.reshape(b * n, d)

    rows, idx3 = pl.pallas_call(
        _assign_kernel,
        grid=(NKC + NTT,),
        in_specs=[
            pl.BlockSpec((KT, DIM), lambda i: (jnp.minimum(i, NKC - 1), 0)),
            pl.BlockSpec((DIM, DIM), lambda i: (0, 0)),
            pl.BlockSpec((TOK, DIM), lambda i: (jnp.maximum(i - NKC, 0), 0)),
        ],
        out_specs=[
            pl.BlockSpec((KT, DIM), lambda i: (jnp.minimum(i, NKC - 1), 0)),
            pl.BlockSpec((1, 1, TOK), lambda i: (jnp.maximum(i - NKC, 0), 0, 0)),
        ],
        out_shape=[
            jax.ShapeDtypeStruct((K, DIM), jnp.float32),
            jax.ShapeDtypeStruct((NTT, 1, TOK), jnp.int32),
        ],
        scratch_shapes=[pltpu.VMEM((NKC, DIM, KT), jnp.float32)],
    )(frozen_codebook, W, xf)

    indices = idx3.reshape(BN)
    quant = _gather_kernel(rows, indices.reshape(NW, NCH, CH))
    qf = quant.reshape(BN, DIM)

    rot, loss = pl.pallas_call(
        _rot_kernel,
        grid=(BN // TOK4,),
        in_specs=[
            pl.BlockSpec((TOK4, DIM), lambda i: (i, 0)),
            pl.BlockSpec((TOK4, DIM), lambda i: (i, 0)),
        ],
        out_specs=[
            pl.BlockSpec((TOK4, DIM), lambda i: (i, 0)),
            pl.BlockSpec((1, 1), lambda i: (0, 0)),
        ],
        out_shape=[
            jax.ShapeDtypeStruct((BN, DIM), jnp.float32),
            jax.ShapeDtypeStruct((1, 1), jnp.float32),
        ],
    )(xf, qf)

    return rot.reshape(b, n, d), indices.reshape(b, n), loss.reshape(())


# speedup vs baseline: 2.6142x; 1.0124x over previous
import functools

import jax
import jax.numpy as jnp
from jax import lax
from jax.experimental import pallas as pl
from jax.experimental.pallas import tpu as pltpu
from jax.experimental.pallas import tpu_sc as plsc

B, N, DIM = 16, 576, 256
BN = B * N
K = 8192

KT = 1024
NKC = K // KT
GPC = KT // 128
TOK = 1024
NTT = BN // TOK
TOK4 = 1152

NC, NS = 2, 16
NW = NC * NS
BPW = BN // NW
NCH, CH = 3, 96

_MINOR = (((1,), (1,)), ((), ()))


def _assign_kernel(cb_ref, w_ref, x_ref, rows_ref, idx_ref, cols_scr):
    i = pl.program_id(0)

    @pl.when(i < NKC)
    def _():
        cb = cb_ref[...]
        w = w_ref[...]
        icb = lax.dot_general(cb, w, _MINOR, preferred_element_type=jnp.float32)
        rn = jnp.sqrt(jnp.sum(icb * icb, axis=1, keepdims=True))
        rows_ref[...] = icb * (1.0 / jnp.clip(rn, 1e-12))
        icbt = lax.dot_general(w, cb, _MINOR, preferred_element_type=jnp.float32)
        cn = jnp.sqrt(jnp.sum(icbt * icbt, axis=0, keepdims=True))
        cols_scr[jnp.minimum(i, NKC - 1)] = icbt / jnp.clip(cn, 1e-12)

    @pl.when(i >= NKC)
    def _():
        xb = x_ref[...]
        nrm = jnp.sqrt(jnp.sum(xb * xb, axis=1, keepdims=True))
        xn = xb / jnp.clip(nrm, 1e-12)

        bv = jnp.full((TOK, 128), -jnp.inf, dtype=jnp.float32)
        bg = jnp.zeros((TOK, 128), dtype=jnp.int32)
        for c in range(NKC):
            sim = jnp.dot(xn, cols_scr[c], preferred_element_type=jnp.float32)
            for g in range(GPC):
                v = sim[:, g * 128:(g + 1) * 128]
                upd = v > bv
                bv = jnp.where(upd, v, bv)
                bg = jnp.where(upd, c * GPC + g, bg)

        lane = lax.broadcasted_iota(jnp.int32, (TOK, 128), 1)
        gidx = bg * 128 + lane
        m = jnp.max(bv, axis=1, keepdims=True)
        cand = jnp.where(bv == m, gidx, K)
        idx_ref[...] = jnp.min(cand, axis=1).reshape(1, 1, TOK)


def _rot_kernel(x_ref, q_ref, out_ref, loss_ref):
    i = pl.program_id(0)
    xb = x_ref[...]
    q = q_ref[...]
    a = jnp.sum(xb * xb, axis=1, keepdims=True)
    bq = jnp.sum(xb * q, axis=1, keepdims=True)
    c = jnp.sum(q * q, axis=1, keepdims=True)
    rinv = 1.0 / jnp.clip(jnp.sqrt(a), 1e-12)
    eu = a * rinv * rinv
    eq = bq * rinv
    ss = eu + 2.0 * eq + c
    winv = 1.0 / jnp.clip(jnp.sqrt(ss), 1e-6)
    eww = (eu + eq) * winv * winv
    alpha = rinv * (1.0 - 2.0 * eww)
    beta = 2.0 * (eu - eww)
    out_ref[...] = alpha * xb + beta * q
    part = (jnp.sum(eu - 2.0 * eq + c, axis=(0, 1), keepdims=True)
            * (1.25 / (BN * DIM)))

    @pl.when(i == 0)
    def _():
        loss_ref[...] = jnp.zeros_like(part)

    loss_ref[...] += part


@functools.lru_cache(maxsize=1)
def _make_gather():
    mesh = plsc.VectorSubcoreMesh(
        core_axis_name="c", subcore_axis_name="s",
        num_cores=NC, num_subcores=NS)

    @functools.partial(
        pl.kernel,
        mesh=mesh,
        out_type=jax.ShapeDtypeStruct((NW, NCH, CH, DIM), jnp.float32),
        scratch_types=[
            pltpu.VMEM((NCH, CH), jnp.int32),
            pltpu.VMEM((NCH, CH, DIM), jnp.float32),
            pltpu.SemaphoreType.DMA,
        ],
    )
    def _gather_body(table_hbm, idx_hbm, out_hbm, idx_v, rows_v, sem):
        wid = lax.axis_index("s") * NC + lax.axis_index("c")
        pltpu.sync_copy(idx_hbm.at[wid], idx_v)
        copies = [
            pltpu.async_copy(table_hbm.at[idx_v.at[j]], rows_v.at[j], sem)
            for j in range(NCH)
        ]
        for c in copies:
            c.wait()
        pltpu.sync_copy(rows_v, out_hbm.at[wid])

    return _gather_body


def _gather_kernel(table, idx3):
    return _make_gather()(table, idx3)


def kernel(x, frozen_codebook, W):
    b, n, d = x.shape
    xf = x.reshape(b * n, d)

    rows, idx3 = pl.pallas_call(
        _assign_kernel,
        grid=(NKC + NTT,),
        in_specs=[
            pl.BlockSpec((KT, DIM), lambda i: (jnp.minimum(i, NKC - 1), 0)),
            pl.BlockSpec((DIM, DIM), lambda i: (0, 0)),
            pl.BlockSpec((TOK, DIM), lambda i: (jnp.maximum(i - NKC, 0), 0)),
        ],
        out_specs=[
            pl.BlockSpec((KT, DIM), lambda i: (jnp.minimum(i, NKC - 1), 0)),
            pl.BlockSpec((1, 1, TOK), lambda i: (jnp.maximum(i - NKC, 0), 0, 0)),
        ],
        out_shape=[
            jax.ShapeDtypeStruct((K, DIM), jnp.float32),
            jax.ShapeDtypeStruct((NTT, 1, TOK), jnp.int32),
        ],
        scratch_shapes=[pltpu.VMEM((NKC, DIM, KT), jnp.float32)],
    )(frozen_codebook, W, xf)

    indices = idx3.reshape(BN)
    quant = _gather_kernel(rows, indices.reshape(NW, NCH, CH))
    qf = quant.reshape(BN, DIM)

    rot, loss = pl.pallas_call(
        _rot_kernel,
        grid=(BN // TOK4,),
        in_specs=[
            pl.BlockSpec((TOK4, DIM), lambda i: (i, 0)),
            pl.BlockSpec((TOK4, DIM), lambda i: (i, 0)),
        ],
        out_specs=[
            pl.BlockSpec((TOK4, DIM), lambda i: (i, 0)),
            pl.BlockSpec((1, 1), lambda i: (0, 0)),
        ],
        out_shape=[
            jax.ShapeDtypeStruct((BN, DIM), jnp.float32),
            jax.ShapeDtypeStruct((1, 1), jnp.float32),
        ],
    )(xf, qf)

    return rot.reshape(b, n, d), indices.reshape(b, n), loss.reshape(())


# speedup vs baseline: 2.6188x; 1.0018x over previous
import functools

import jax
import jax.numpy as jnp
from jax import lax
from jax.experimental import pallas as pl
from jax.experimental.pallas import tpu as pltpu
from jax.experimental.pallas import tpu_sc as plsc

B, N, DIM = 16, 576, 256
BN = B * N
K = 8192

KT = 1024
NKC = K // KT
GPC = KT // 128
TOK = 2304
NTT = BN // TOK
TOK4 = 1152

NC, NS = 2, 16
NW = NC * NS
BPW = BN // NW
NCH, CH = 3, 96

_MINOR = (((1,), (1,)), ((), ()))


def _assign_kernel(cb_ref, w_ref, x_ref, rows_ref, idx_ref, cols_scr):
    i = pl.program_id(0)

    @pl.when(i < NKC)
    def _():
        cb = cb_ref[...]
        w = w_ref[...]
        icb = lax.dot_general(cb, w, _MINOR, preferred_element_type=jnp.float32)
        rn = jnp.sqrt(jnp.sum(icb * icb, axis=1, keepdims=True))
        rows_ref[...] = icb * (1.0 / jnp.clip(rn, 1e-12))
        icbt = lax.dot_general(w, cb, _MINOR, preferred_element_type=jnp.float32)
        cn = jnp.sqrt(jnp.sum(icbt * icbt, axis=0, keepdims=True))
        cols_scr[jnp.minimum(i, NKC - 1)] = icbt / jnp.clip(cn, 1e-12)

    @pl.when(i >= NKC)
    def _():
        xb = x_ref[...]
        nrm = jnp.sqrt(jnp.sum(xb * xb, axis=1, keepdims=True))
        xn = xb / jnp.clip(nrm, 1e-12)

        bv = jnp.full((TOK, 128), -jnp.inf, dtype=jnp.float32)
        bg = jnp.zeros((TOK, 128), dtype=jnp.int32)
        for c in range(NKC):
            sim = jnp.dot(xn, cols_scr[c], preferred_element_type=jnp.float32)
            for g in range(GPC):
                v = sim[:, g * 128:(g + 1) * 128]
                upd = v > bv
                bv = jnp.where(upd, v, bv)
                bg = jnp.where(upd, c * GPC + g, bg)

        lane = lax.broadcasted_iota(jnp.int32, (TOK, 128), 1)
        gidx = bg * 128 + lane
        m = jnp.max(bv, axis=1, keepdims=True)
        cand = jnp.where(bv == m, gidx, K)
        idx_ref[...] = jnp.min(cand, axis=1).reshape(1, 1, TOK)


def _rot_kernel(x_ref, q_ref, out_ref, loss_ref):
    i = pl.program_id(0)
    xb = x_ref[...]
    q = q_ref[...]
    a = jnp.sum(xb * xb, axis=1, keepdims=True)
    bq = jnp.sum(xb * q, axis=1, keepdims=True)
    c = jnp.sum(q * q, axis=1, keepdims=True)
    rinv = 1.0 / jnp.clip(jnp.sqrt(a), 1e-12)
    eu = a * rinv * rinv
    eq = bq * rinv
    ss = eu + 2.0 * eq + c
    winv = 1.0 / jnp.clip(jnp.sqrt(ss), 1e-6)
    eww = (eu + eq) * winv * winv
    alpha = rinv * (1.0 - 2.0 * eww)
    beta = 2.0 * (eu - eww)
    out_ref[...] = alpha * xb + beta * q
    part = (jnp.sum(eu - 2.0 * eq + c, axis=(0, 1), keepdims=True)
            * (1.25 / (BN * DIM)))

    @pl.when(i == 0)
    def _():
        loss_ref[...] = jnp.zeros_like(part)

    loss_ref[...] += part


@functools.lru_cache(maxsize=1)
def _make_gather():
    mesh = plsc.VectorSubcoreMesh(
        core_axis_name="c", subcore_axis_name="s",
        num_cores=NC, num_subcores=NS)

    @functools.partial(
        pl.kernel,
        mesh=mesh,
        out_type=jax.ShapeDtypeStruct((NW, NCH, CH, DIM), jnp.float32),
        scratch_types=[
            pltpu.VMEM((NCH, CH), jnp.int32),
            pltpu.VMEM((NCH, CH, DIM), jnp.float32),
            pltpu.SemaphoreType.DMA,
        ],
    )
    def _gather_body(table_hbm, idx_hbm, out_hbm, idx_v, rows_v, sem):
        wid = lax.axis_index("s") * NC + lax.axis_index("c")
        pltpu.sync_copy(idx_hbm.at[wid], idx_v)
        copies = [
            pltpu.async_copy(table_hbm.at[idx_v.at[j]], rows_v.at[j], sem)
            for j in range(NCH)
        ]
        for c in copies:
            c.wait()
        pltpu.sync_copy(rows_v, out_hbm.at[wid])

    return _gather_body


def _gather_kernel(table, idx3):
    return _make_gather()(table, idx3)


def kernel(x, frozen_codebook, W):
    b, n, d = x.shape
    xf = x.reshape(b * n, d)

    rows, idx3 = pl.pallas_call(
        _assign_kernel,
        grid=(NKC + NTT,),
        in_specs=[
            pl.BlockSpec((KT, DIM), lambda i: (jnp.minimum(i, NKC - 1), 0)),
            pl.BlockSpec((DIM, DIM), lambda i: (0, 0)),
            pl.BlockSpec((TOK, DIM), lambda i: (jnp.maximum(i - NKC, 0), 0)),
        ],
        out_specs=[
            pl.BlockSpec((KT, DIM), lambda i: (jnp.minimum(i, NKC - 1), 0)),
            pl.BlockSpec((1, 1, TOK), lambda i: (jnp.maximum(i - NKC, 0), 0, 0)),
        ],
        out_shape=[
            jax.ShapeDtypeStruct((K, DIM), jnp.float32),
            jax.ShapeDtypeStruct((NTT, 1, TOK), jnp.int32),
        ],
        scratch_shapes=[pltpu.VMEM((NKC, DIM, KT), jnp.float32)],
    )(frozen_codebook, W, xf)

    indices = idx3.reshape(BN)
    quant = _gather_kernel(rows, indices.reshape(NW, NCH, CH))
    qf = quant.reshape(BN, DIM)

    rot, loss = pl.pallas_call(
        _rot_kernel,
        grid=(BN // TOK4,),
        in_specs=[
            pl.BlockSpec((TOK4, DIM), lambda i: (i, 0)),
            pl.BlockSpec((TOK4, DIM), lambda i: (i, 0)),
        ],
        out_specs=[
            pl.BlockSpec((TOK4, DIM), lambda i: (i, 0)),
            pl.BlockSpec((1, 1), lambda i: (0, 0)),
        ],
        out_shape=[
            jax.ShapeDtypeStruct((BN, DIM), jnp.float32),
            jax.ShapeDtypeStruct((1, 1), jnp.float32),
        ],
    )(xf, qf)

    return rot.reshape(b, n, d), indices.reshape(b, n), loss.reshape(())


# speedup vs baseline: 2.6667x; 1.0183x over previous
import functools

import jax
import jax.numpy as jnp
from jax import lax
from jax.experimental import pallas as pl
from jax.experimental.pallas import tpu as pltpu
from jax.experimental.pallas import tpu_sc as plsc

B, N, DIM = 16, 576, 256
BN = B * N
K = 8192

KT = 2048
NKC = K // KT
GPC = KT // 128
TOK = 2304
NTT = BN // TOK
TOK4 = 1152

NC, NS = 2, 16
NW = NC * NS
BPW = BN // NW
NCH, CH = 3, 96

_MINOR = (((1,), (1,)), ((), ()))


def _assign_kernel(cb_ref, w_ref, x_ref, rows_ref, idx_ref, cols_scr):
    i = pl.program_id(0)

    @pl.when(i < NKC)
    def _():
        cb = cb_ref[...]
        w = w_ref[...]
        icb = lax.dot_general(cb, w, _MINOR, preferred_element_type=jnp.float32)
        rn = jnp.sqrt(jnp.sum(icb * icb, axis=1, keepdims=True))
        rows_ref[...] = icb * (1.0 / jnp.clip(rn, 1e-12))
        icbt = lax.dot_general(w, cb, _MINOR, preferred_element_type=jnp.float32)
        cn = jnp.sqrt(jnp.sum(icbt * icbt, axis=0, keepdims=True))
        cols_scr[jnp.minimum(i, NKC - 1)] = icbt / jnp.clip(cn, 1e-12)

    @pl.when(i >= NKC)
    def _():
        xb = x_ref[...]
        nrm = jnp.sqrt(jnp.sum(xb * xb, axis=1, keepdims=True))
        xn = xb / jnp.clip(nrm, 1e-12)

        bv = jnp.full((TOK, 128), -jnp.inf, dtype=jnp.float32)
        bg = jnp.zeros((TOK, 128), dtype=jnp.int32)
        for c in range(NKC):
            sim = jnp.dot(xn, cols_scr[c], preferred_element_type=jnp.float32)
            for g in range(GPC):
                v = sim[:, g * 128:(g + 1) * 128]
                upd = v > bv
                bv = jnp.where(upd, v, bv)
                bg = jnp.where(upd, c * GPC + g, bg)

        lane = lax.broadcasted_iota(jnp.int32, (TOK, 128), 1)
        gidx = bg * 128 + lane
        m = jnp.max(bv, axis=1, keepdims=True)
        cand = jnp.where(bv == m, gidx, K)
        idx_ref[...] = jnp.min(cand, axis=1).reshape(1, 1, TOK)


def _rot_kernel(x_ref, q_ref, out_ref, loss_ref):
    i = pl.program_id(0)
    xb = x_ref[...]
    q = q_ref[...]
    a = jnp.sum(xb * xb, axis=1, keepdims=True)
    bq = jnp.sum(xb * q, axis=1, keepdims=True)
    c = jnp.sum(q * q, axis=1, keepdims=True)
    rinv = 1.0 / jnp.clip(jnp.sqrt(a), 1e-12)
    eu = a * rinv * rinv
    eq = bq * rinv
    ss = eu + 2.0 * eq + c
    winv = 1.0 / jnp.clip(jnp.sqrt(ss), 1e-6)
    eww = (eu + eq) * winv * winv
    alpha = rinv * (1.0 - 2.0 * eww)
    beta = 2.0 * (eu - eww)
    out_ref[...] = alpha * xb + beta * q
    part = (jnp.sum(eu - 2.0 * eq + c, axis=(0, 1), keepdims=True)
            * (1.25 / (BN * DIM)))

    @pl.when(i == 0)
    def _():
        loss_ref[...] = jnp.zeros_like(part)

    loss_ref[...] += part


@functools.lru_cache(maxsize=1)
def _make_gather():
    mesh = plsc.VectorSubcoreMesh(
        core_axis_name="c", subcore_axis_name="s",
        num_cores=NC, num_subcores=NS)

    @functools.partial(
        pl.kernel,
        mesh=mesh,
        out_type=jax.ShapeDtypeStruct((NW, NCH, CH, DIM), jnp.float32),
        scratch_types=[
            pltpu.VMEM((NCH, CH), jnp.int32),
            pltpu.VMEM((NCH, CH, DIM), jnp.float32),
            pltpu.SemaphoreType.DMA,
        ],
    )
    def _gather_body(table_hbm, idx_hbm, out_hbm, idx_v, rows_v, sem):
        wid = lax.axis_index("s") * NC + lax.axis_index("c")
        pltpu.sync_copy(idx_hbm.at[wid], idx_v)
        copies = [
            pltpu.async_copy(table_hbm.at[idx_v.at[j]], rows_v.at[j], sem)
            for j in range(NCH)
        ]
        for c in copies:
            c.wait()
        pltpu.sync_copy(rows_v, out_hbm.at[wid])

    return _gather_body


def _gather_kernel(table, idx3):
    return _make_gather()(table, idx3)


def kernel(x, frozen_codebook, W):
    b, n, d = x.shape
    xf = x.reshape(b * n, d)

    rows, idx3 = pl.pallas_call(
        _assign_kernel,
        grid=(NKC + NTT,),
        in_specs=[
            pl.BlockSpec((KT, DIM), lambda i: (jnp.minimum(i, NKC - 1), 0)),
            pl.BlockSpec((DIM, DIM), lambda i: (0, 0)),
            pl.BlockSpec((TOK, DIM), lambda i: (jnp.maximum(i - NKC, 0), 0)),
        ],
        out_specs=[
            pl.BlockSpec((KT, DIM), lambda i: (jnp.minimum(i, NKC - 1), 0)),
            pl.BlockSpec((1, 1, TOK), lambda i: (jnp.maximum(i - NKC, 0), 0, 0)),
        ],
        out_shape=[
            jax.ShapeDtypeStruct((K, DIM), jnp.float32),
            jax.ShapeDtypeStruct((NTT, 1, TOK), jnp.int32),
        ],
        scratch_shapes=[pltpu.VMEM((NKC, DIM, KT), jnp.float32)],
    )(frozen_codebook, W, xf)

    indices = idx3.reshape(BN)
    quant = _gather_kernel(rows, indices.reshape(NW, NCH, CH))
    qf = quant.reshape(BN, DIM)

    rot, loss = pl.pallas_call(
        _rot_kernel,
        grid=(BN // TOK4,),
        in_specs=[
            pl.BlockSpec((TOK4, DIM), lambda i: (i, 0)),
            pl.BlockSpec((TOK4, DIM), lambda i: (i, 0)),
        ],
        out_specs=[
            pl.BlockSpec((TOK4, DIM), lambda i: (i, 0)),
            pl.BlockSpec((1, 1), lambda i: (0, 0)),
        ],
        out_shape=[
            jax.ShapeDtypeStruct((BN, DIM), jnp.float32),
            jax.ShapeDtypeStruct((1, 1), jnp.float32),
        ],
    )(xf, qf)

    return rot.reshape(b, n, d), indices.reshape(b, n), loss.reshape(())
